# baseline (speedup 1.0000x reference)
import functools

import jax
import jax.numpy as jnp
from jax import lax
from jax.experimental import pallas as pl
from jax.experimental.pallas import tpu as pltpu
from jax.experimental.pallas import tpu_sc as plsc

_TOP_K = 8
_N_GROUPS = 8
_GROUP_SIZE = 8
_LANES = 16

_SORT8 = (
    (0, 1), (2, 3), (4, 5), (6, 7),
    (0, 2), (1, 3), (4, 6), (5, 7),
    (1, 2), (5, 6), (0, 4), (3, 7),
    (1, 5), (2, 6),
    (1, 4), (3, 6),
    (2, 4), (3, 5),
    (3, 4),
)
_BMERGE8 = (
    (0, 4), (1, 5), (2, 6), (3, 7),
    (0, 2), (1, 3), (4, 6), (5, 7),
    (0, 1), (2, 3), (4, 5), (6, 7),
)


def _gate_softmax_body(x_ref, w_ref, p_ref):
    s = lax.dot_general(
        w_ref[...], x_ref[...],
        (((1,), (1,)), ((), ())),
        preferred_element_type=jnp.float32,
    )
    m = jnp.max(s, axis=0, keepdims=True)
    e = jnp.exp(s - m)
    p_ref[0] = e / jnp.sum(e, axis=0, keepdims=True)


def _gate_softmax(x, w_gate, nw, tok_w, block=512):
    n, d = x.shape
    n_e = w_gate.shape[0]
    s = tok_w // block
    return pl.pallas_call(
        _gate_softmax_body,
        grid=(n // block,),
        in_specs=[
            pl.BlockSpec((block, d), lambda i: (i, 0)),
            pl.BlockSpec((n_e, d), lambda i: (0, 0)),
        ],
        out_specs=pl.BlockSpec((1, n_e, block), lambda i: (i // s, 0, i % s)),
        out_shape=jax.ShapeDtypeStruct((nw, n_e, tok_w), jnp.float32),
    )(x, w_gate)


def _lex_gt(av, ai, bv, bi):
    return (av > bv) | ((av == bv) & (ai < bi))


def _ce(vals, idxs, a, b):
    c = _lex_gt(vals[a], idxs[a], vals[b], idxs[b])
    va = jnp.where(c, vals[a], vals[b])
    vb = jnp.where(c, vals[b], vals[a])
    ia = jnp.where(c, idxs[a], idxs[b])
    ib = jnp.where(c, idxs[b], idxs[a])
    vals[a], vals[b], idxs[a], idxs[b] = va, vb, ia, ib


def _route_sc(p3, n, n_e):
    info = plsc.get_sparse_core_info()
    nw = info.num_cores * info.num_subcores
    tok_w = n // nw
    n_chunks = tok_w // _LANES
    mesh = plsc.VectorSubcoreMesh(core_axis_name="c", subcore_axis_name="s")
    p_flat = p3.reshape(nw * n_e * tok_w)

    @functools.partial(
        pl.kernel,
        mesh=mesh,
        compiler_params=pltpu.CompilerParams(needs_layout_passes=False),
        out_type=[
            jax.ShapeDtypeStruct((n * _TOP_K,), jnp.float32),
            jax.ShapeDtypeStruct((n * _TOP_K,), jnp.int32),
            jax.ShapeDtypeStruct((nw, n_e), jnp.float32),
        ],
        scratch_types=[
            pltpu.VMEM((n_e * tok_w,), jnp.float32),
            pltpu.VMEM((tok_w * _TOP_K,), jnp.float32),
            pltpu.VMEM((tok_w * _TOP_K,), jnp.int32),
            pltpu.VMEM((n_e,), jnp.float32),
        ],
    )
    def body(pt_hbm, w_hbm, idx_hbm, cnt_hbm, p_v, wout_v, iout_v, cnt_v):
        cid = lax.axis_index("c")
        sid = lax.axis_index("s")
        wid = sid * info.num_cores + cid
        base = wid * tok_w
        pltpu.sync_copy(pt_hbm.at[pl.ds(wid * n_e * tok_w, n_e * tok_w)], p_v)
        zeros = jnp.zeros((_LANES,), jnp.float32)
        for j in range(n_e // _LANES):
            cnt_v[pl.ds(j * _LANES, _LANES)] = zeros
        lanes = lax.iota(jnp.int32, _LANES)
        ones = jnp.ones((_LANES,), jnp.float32)

        def chunk(c, carry):
            col = c * _LANES
            rows = col + lanes
            g = []
            for j in range(_N_GROUPS):
                m = p_v[pl.ds(j * _GROUP_SIZE * tok_w + col, _LANES)]
                for o in range(1, _GROUP_SIZE):
                    m = jnp.maximum(
                        m,
                        p_v[pl.ds((j * _GROUP_SIZE + o) * tok_w + col,
                                  _LANES)])
                g.append(m)
            bv, bi = g[0], jnp.zeros((_LANES,), jnp.int32)
            for j in range(1, _N_GROUPS):
                c1 = g[j] > bv
                bv = jnp.where(c1, g[j], bv)
                bi = jnp.where(c1, j, bi)
            sv = jnp.full((_LANES,), -jnp.inf, jnp.float32)
            si = jnp.zeros((_LANES,), jnp.int32)
            for j in range(_N_GROUPS):
                c2 = (bi != j) & (g[j] > sv)
                sv = jnp.where(c2, g[j], sv)
                si = jnp.where(c2, j, si)
            a_v, a_i, b_v, b_i = [], [], [], []
            for o in range(_GROUP_SIZE):
                ia = bi * _GROUP_SIZE + o
                ib = si * _GROUP_SIZE + o
                a_i.append(ia)
                b_i.append(ib)
                a_v.append(plsc.load_gather(p_v, [ia * tok_w + rows]))
                b_v.append(plsc.load_gather(p_v, [ib * tok_w + rows]))
            for aa, bb in _SORT8:
                _ce(a_v, a_i, aa, bb)
                _ce(b_v, b_i, aa, bb)
            l_v, l_i = [], []
            for k in range(_TOP_K):
                c3 = _lex_gt(a_v[k], a_i[k], b_v[7 - k], b_i[7 - k])
                l_v.append(jnp.where(c3, a_v[k], b_v[7 - k]))
                l_i.append(jnp.where(c3, a_i[k], b_i[7 - k]))
            for aa, bb in _BMERGE8:
                _ce(l_v, l_i, aa, bb)
            out_pos = rows * _TOP_K
            for k in range(_TOP_K):
                plsc.store_scatter(wout_v, [out_pos + k], l_v[k])
                plsc.store_scatter(iout_v, [out_pos + k], l_i[k])
                plsc.addupdate_scatter(cnt_v, [l_i[k]], ones)
            return carry

        lax.fori_loop(0, n_chunks, chunk, 0)
        pltpu.sync_copy(wout_v, w_hbm.at[pl.ds(base * _TOP_K, tok_w * _TOP_K)])
        pltpu.sync_copy(iout_v,
                        idx_hbm.at[pl.ds(base * _TOP_K, tok_w * _TOP_K)])
        pltpu.sync_copy(cnt_v, cnt_hbm.at[wid])

    return body(p_flat)


def kernel(x, w_gate):
    n = x.shape[0]
    n_e = w_gate.shape[0]
    info = plsc.get_sparse_core_info()
    nw = info.num_cores * info.num_subcores
    p3 = _gate_softmax(x, w_gate, nw, n // nw)
    w_flat, idx_flat, cnt_parts = _route_sc(p3, n, n_e)
    weights = w_flat.reshape(n, _TOP_K).astype(x.dtype)
    topk_indices = idx_flat.reshape(n, _TOP_K)
    counts = jnp.sum(cnt_parts, axis=0)
    return (weights, topk_indices, counts)


# speedup vs baseline: 1.0688x; 1.0688x over previous
import functools

import jax
import jax.numpy as jnp
from jax import lax
from jax.experimental import pallas as pl
from jax.experimental.pallas import tpu as pltpu
from jax.experimental.pallas import tpu_sc as plsc

_TOP_K = 8
_N_GROUPS = 8
_GROUP_SIZE = 8
_LANES = 16

_SORT8 = (
    (0, 1), (2, 3), (4, 5), (6, 7),
    (0, 2), (1, 3), (4, 6), (5, 7),
    (1, 2), (5, 6), (0, 4), (3, 7),
    (1, 5), (2, 6),
    (1, 4), (3, 6),
    (2, 4), (3, 5),
    (3, 4),
)
_BMERGE8 = (
    (0, 4), (1, 5), (2, 6), (3, 7),
    (0, 2), (1, 3), (4, 6), (5, 7),
    (0, 1), (2, 3), (4, 5), (6, 7),
)


def _gate_softmax_body(x_ref, w_ref, p_ref):
    s = lax.dot_general(
        w_ref[...], x_ref[...],
        (((1,), (1,)), ((), ())),
        preferred_element_type=jnp.float32,
    )
    m = jnp.max(s, axis=0, keepdims=True)
    e = jnp.exp(s - m)
    p = e / jnp.sum(e, axis=0, keepdims=True)
    p_ref[...] = p.reshape(p_ref.shape)


def _gate_softmax(x, w_gate, nw, tok_w, block=1024):
    n, d = x.shape
    n_e = w_gate.shape[0]
    del tok_w, nw
    rows_per_block = n_e * block // 128
    return pl.pallas_call(
        _gate_softmax_body,
        grid=(n // block,),
        in_specs=[
            pl.BlockSpec((block, d), lambda i: (i, 0)),
            pl.BlockSpec((n_e, d), lambda i: (0, 0)),
        ],
        out_specs=pl.BlockSpec((rows_per_block, 128), lambda i: (i, 0)),
        out_shape=jax.ShapeDtypeStruct((n_e * n // 128, 128), jnp.float32),
    )(x, w_gate)


def _lex_gt(av, ai, bv, bi):
    return (av > bv) | ((av == bv) & (ai < bi))


def _ce(vals, idxs, a, b):
    c = _lex_gt(vals[a], idxs[a], vals[b], idxs[b])
    va = jnp.where(c, vals[a], vals[b])
    vb = jnp.where(c, vals[b], vals[a])
    ia = jnp.where(c, idxs[a], idxs[b])
    ib = jnp.where(c, idxs[b], idxs[a])
    vals[a], vals[b], idxs[a], idxs[b] = va, vb, ia, ib


def _route_sc(p3, n, n_e):
    info = plsc.get_sparse_core_info()
    nw = info.num_cores * info.num_subcores
    tok_w = n // nw
    n_chunks = tok_w // _LANES
    mesh = plsc.VectorSubcoreMesh(core_axis_name="c", subcore_axis_name="s")
    p_flat = p3.reshape(n * n_e)

    @functools.partial(
        pl.kernel,
        mesh=mesh,
        compiler_params=pltpu.CompilerParams(needs_layout_passes=False),
        out_type=[
            jax.ShapeDtypeStruct((n * _TOP_K,), jnp.float32),
            jax.ShapeDtypeStruct((n * _TOP_K,), jnp.int32),
            jax.ShapeDtypeStruct((nw, n_e), jnp.float32),
        ],
        scratch_types=[
            pltpu.VMEM((n_e * tok_w,), jnp.float32),
            pltpu.VMEM((tok_w * _TOP_K,), jnp.float32),
            pltpu.VMEM((tok_w * _TOP_K,), jnp.int32),
            pltpu.VMEM((n_e,), jnp.float32),
        ],
    )
    def body(pt_hbm, w_hbm, idx_hbm, cnt_hbm, p_v, wout_v, iout_v, cnt_v):
        cid = lax.axis_index("c")
        sid = lax.axis_index("s")
        wid = sid * info.num_cores + cid
        base = wid * tok_w
        pltpu.sync_copy(pt_hbm.at[pl.ds(wid * n_e * tok_w, n_e * tok_w)], p_v)
        zeros = jnp.zeros((_LANES,), jnp.float32)
        for j in range(n_e // _LANES):
            cnt_v[pl.ds(j * _LANES, _LANES)] = zeros
        lanes = lax.iota(jnp.int32, _LANES)
        ones = jnp.ones((_LANES,), jnp.float32)

        def chunk(c, carry):
            col = c * _LANES
            rows = col + lanes
            g = []
            for j in range(_N_GROUPS):
                m = p_v[pl.ds(j * _GROUP_SIZE * tok_w + col, _LANES)]
                for o in range(1, _GROUP_SIZE):
                    m = jnp.maximum(
                        m,
                        p_v[pl.ds((j * _GROUP_SIZE + o) * tok_w + col,
                                  _LANES)])
                g.append(m)
            bv, bi = g[0], jnp.zeros((_LANES,), jnp.int32)
            for j in range(1, _N_GROUPS):
                c1 = g[j] > bv
                bv = jnp.where(c1, g[j], bv)
                bi = jnp.where(c1, j, bi)
            sv = jnp.full((_LANES,), -jnp.inf, jnp.float32)
            si = jnp.zeros((_LANES,), jnp.int32)
            for j in range(_N_GROUPS):
                c2 = (bi != j) & (g[j] > sv)
                sv = jnp.where(c2, g[j], sv)
                si = jnp.where(c2, j, si)
            a_v, a_i, b_v, b_i = [], [], [], []
            for o in range(_GROUP_SIZE):
                ia = bi * _GROUP_SIZE + o
                ib = si * _GROUP_SIZE + o
                a_i.append(ia)
                b_i.append(ib)
                a_v.append(plsc.load_gather(p_v, [ia * tok_w + rows]))
                b_v.append(plsc.load_gather(p_v, [ib * tok_w + rows]))
            for aa, bb in _SORT8:
                _ce(a_v, a_i, aa, bb)
                _ce(b_v, b_i, aa, bb)
            l_v, l_i = [], []
            for k in range(_TOP_K):
                c3 = _lex_gt(a_v[k], a_i[k], b_v[7 - k], b_i[7 - k])
                l_v.append(jnp.where(c3, a_v[k], b_v[7 - k]))
                l_i.append(jnp.where(c3, a_i[k], b_i[7 - k]))
            for aa, bb in _BMERGE8:
                _ce(l_v, l_i, aa, bb)
            out_pos = rows * _TOP_K
            for k in range(_TOP_K):
                plsc.store_scatter(wout_v, [out_pos + k], l_v[k])
                plsc.store_scatter(iout_v, [out_pos + k], l_i[k])
                plsc.addupdate_scatter(cnt_v, [l_i[k]], ones)
            return carry

        lax.fori_loop(0, n_chunks, chunk, 0)
        pltpu.sync_copy(wout_v, w_hbm.at[pl.ds(base * _TOP_K, tok_w * _TOP_K)])
        pltpu.sync_copy(iout_v,
                        idx_hbm.at[pl.ds(base * _TOP_K, tok_w * _TOP_K)])
        pltpu.sync_copy(cnt_v, cnt_hbm.at[wid])

    return body(p_flat)


def kernel(x, w_gate):
    n = x.shape[0]
    n_e = w_gate.shape[0]
    info = plsc.get_sparse_core_info()
    nw = info.num_cores * info.num_subcores
    p3 = _gate_softmax(x, w_gate, nw, n // nw)
    w_flat, idx_flat, cnt_parts = _route_sc(p3, n, n_e)
    weights = w_flat.reshape(n, _TOP_K).astype(x.dtype)
    topk_indices = idx_flat.reshape(n, _TOP_K)
    counts = jnp.sum(cnt_parts, axis=0)
    return (weights, topk_indices, counts)
